# baseline (speedup 1.0000x reference)
import jax
import jax.numpy as jnp
from jax.experimental import pallas as pl
from jax.experimental.pallas import tpu as pltpu

BS = 8
NC = 16
C = 128
D = 64
K = 16
K_B = 8
B_BORDER = 16
H_MOD = 32
N = NC * C
MOD_IN = K + 3 * D + 1
MOD_OUT = K + K_B + 1 + D
G1 = 8


def _cell_kernel(heb_ref, h_ref, dec_ref, prim_ref, w1_ref, b1_ref, w2_ref,
                 b2_ref, nid_ref, idx_ref,
                 wconn_ref, sel_ref, ndec_ref, nprim_ref):
    heb = heb_ref[:, 0]
    hh = h_ref[:, 0]
    dec = dec_ref[...]
    prim = prim_ref[:, 0]
    nid = nid_ref[0]
    nid_b = jnp.broadcast_to(nid[None], (BS, C, D))
    x = jnp.concatenate([heb, hh, dec[..., None], prim, nid_b], axis=-1)

    w1 = w1_ref[...]
    b1 = b1_ref[...]
    xb = x.astype(jnp.bfloat16)
    hid_parts = []
    for g in range(C // G1):
        xg = jnp.swapaxes(xb[:, G1 * g:G1 * (g + 1), :], 0, 1)
        a = xg.reshape(G1 * BS, MOD_IN)
        wg = w1[G1 * g:G1 * (g + 1)].reshape(G1 * H_MOD, MOD_IN)
        hg = jax.lax.dot_general(
            a, wg,
            dimension_numbers=(((1,), (1,)), ((), ())),
            preferred_element_type=jnp.float32)
        tiles = [hg[BS * q:BS * (q + 1), H_MOD * q:H_MOD * (q + 1)]
                 for q in range(G1)]
        hid_parts.append(jnp.stack(tiles))
    hid = jnp.concatenate(hid_parts, axis=0)
    hid = jnp.tanh(hid + b1[:, None, :])

    w2 = w2_ref[...]
    b2 = b2_ref[...]
    out = jax.lax.dot_general(
        hid.astype(jnp.bfloat16), w2,
        dimension_numbers=(((2,), (1,)), ((0,), (0,))),
        preferred_element_type=jnp.float32)
    out = out + b2[:, None, :]
    out_t = jnp.transpose(out, (1, 0, 2))
    wconn_ref[:, 0] = out_t[:, :, :K]
    ndec_ref[...] = out_t[:, :, K + K_B]
    nprim_ref[:, 0] = out_t[:, :, K + K_B + 1:]

    idx = idx_ref[0, 0]
    iota = jax.lax.broadcasted_iota(jnp.int32, (B_BORDER, C), 1)
    onehot = (idx[:, None] == iota).astype(jnp.float32)
    border = out_t[:, :, K:K + K_B]
    sel_bkj = jax.lax.dot_general(
        border, onehot,
        dimension_numbers=(((1,), (1,)), ((), ())),
        preferred_element_type=jnp.float32)
    sel_ref[:, 0] = jnp.transpose(sel_bkj, (0, 2, 1))


def kernel(h, hebbian_traces, decay_logit, primitives, mod_w1, mod_b1,
           mod_w2, mod_b2, neuron_id, border_indices):
    bidx = border_indices.reshape(NC, 1, B_BORDER).astype(jnp.int32)

    wconn, sel, ndec, nprim = pl.pallas_call(
        _cell_kernel,
        grid=(NC,),
        in_specs=[
            pl.BlockSpec((BS, 1, C, K), lambda c: (0, c, 0, 0)),
            pl.BlockSpec((BS, 1, C, D), lambda c: (0, c, 0, 0)),
            pl.BlockSpec((BS, C), lambda c: (0, c)),
            pl.BlockSpec((BS, 1, C, D), lambda c: (0, c, 0, 0)),
            pl.BlockSpec((C, H_MOD, MOD_IN), lambda c: (c, 0, 0)),
            pl.BlockSpec((C, H_MOD), lambda c: (c, 0)),
            pl.BlockSpec((C, H_MOD, MOD_OUT), lambda c: (c, 0, 0)),
            pl.BlockSpec((C, MOD_OUT), lambda c: (c, 0)),
            pl.BlockSpec((1, C, D), lambda c: (c, 0, 0)),
            pl.BlockSpec((1, 1, B_BORDER), lambda c: (c, 0, 0)),
        ],
        out_specs=[
            pl.BlockSpec((BS, 1, C, K), lambda c: (0, c, 0, 0)),
            pl.BlockSpec((BS, 1, B_BORDER, K_B), lambda c: (0, c, 0, 0)),
            pl.BlockSpec((BS, C), lambda c: (0, c)),
            pl.BlockSpec((BS, 1, C, D), lambda c: (0, c, 0, 0)),
        ],
        out_shape=[
            jax.ShapeDtypeStruct((BS, NC, C, K), jnp.float32),
            jax.ShapeDtypeStruct((BS, NC, B_BORDER, K_B), jnp.float32),
            jax.ShapeDtypeStruct((BS, N), jnp.float32),
            jax.ShapeDtypeStruct((BS, NC, C, D), jnp.float32),
        ],
        compiler_params=pltpu.CompilerParams(
            dimension_semantics=("arbitrary",),
        ),
    )(hebbian_traces, h, decay_logit.reshape(BS, N), primitives,
      mod_w1.astype(jnp.bfloat16), mod_b1,
      mod_w2.astype(jnp.bfloat16), mod_b2, neuron_id, bidx)

    return (wconn, sel, ndec.reshape(BS, NC, C), nprim)


# speedup vs baseline: 1.2204x; 1.2204x over previous
import jax
import jax.numpy as jnp
from jax.experimental import pallas as pl
from jax.experimental.pallas import tpu as pltpu

BS = 8
NC = 16
C = 128
D = 64
K = 16
K_B = 8
B_BORDER = 16
H_MOD = 32
N = NC * C
MOD_IN = K + 3 * D + 1
MOD_OUT = K + K_B + 1 + D
G1 = 8


def _cell_kernel(heb_ref, h_ref, dec_ref, prim_ref, w1_ref, b1_ref, w2_ref,
                 b2_ref, nid_ref, idx_ref,
                 wconn_ref, sel_ref, ndec_ref, nprim_ref):
    heb = heb_ref[:, 0]
    hh = h_ref[:, 0]
    dec = dec_ref[...]
    prim = prim_ref[:, 0]
    nid = nid_ref[0]
    nid_b = jnp.broadcast_to(nid[None], (BS, C, D))
    x = jnp.concatenate([heb, hh, dec[..., None], prim, nid_b], axis=-1)

    w1 = w1_ref[0]
    b1 = b1_ref[...]
    xb = x.astype(jnp.bfloat16)
    hid_parts = []
    for g in range(C // G1):
        xg = jnp.swapaxes(xb[:, G1 * g:G1 * (g + 1), :], 0, 1)
        a = xg.reshape(G1 * BS, MOD_IN)
        wg = w1[G1 * g:G1 * (g + 1)].reshape(G1 * H_MOD, MOD_IN)
        hg = jax.lax.dot_general(
            a, wg,
            dimension_numbers=(((1,), (1,)), ((), ())),
            preferred_element_type=jnp.float32)
        tiles = [hg[BS * q:BS * (q + 1), H_MOD * q:H_MOD * (q + 1)]
                 for q in range(G1)]
        hid_parts.append(jnp.stack(tiles))
    hid = jnp.concatenate(hid_parts, axis=0)
    hid = jnp.tanh(hid + b1[:, None, :])

    w2 = w2_ref[0]
    b2 = b2_ref[...]
    out = jax.lax.dot_general(
        hid.astype(jnp.bfloat16), w2,
        dimension_numbers=(((2,), (1,)), ((0,), (0,))),
        preferred_element_type=jnp.float32)
    out = out + b2[:, None, :]
    out_t = jnp.transpose(out, (1, 0, 2))
    wconn_ref[:, 0] = out_t[:, :, :K]
    ndec_ref[...] = out_t[:, :, K + K_B]
    nprim_ref[:, 0] = out_t[:, :, K + K_B + 1:]

    idx = idx_ref[0, 0]
    iota = jax.lax.broadcasted_iota(jnp.int32, (B_BORDER, C), 1)
    onehot = (idx[:, None] == iota).astype(jnp.float32)
    border = out_t[:, :, K:K + K_B]
    sel_bkj = jax.lax.dot_general(
        border, onehot,
        dimension_numbers=(((1,), (1,)), ((), ())),
        preferred_element_type=jnp.float32)
    sel_ref[:, 0] = jnp.transpose(sel_bkj, (0, 2, 1))


def kernel(h, hebbian_traces, decay_logit, primitives, mod_w1, mod_b1,
           mod_w2, mod_b2, neuron_id, border_indices):
    bidx = border_indices.reshape(NC, 1, B_BORDER).astype(jnp.int32)

    wconn, sel, ndec, nprim = pl.pallas_call(
        _cell_kernel,
        grid=(NC,),
        in_specs=[
            pl.BlockSpec((BS, 1, C, K), lambda c: (0, c, 0, 0)),
            pl.BlockSpec((BS, 1, C, D), lambda c: (0, c, 0, 0)),
            pl.BlockSpec((BS, C), lambda c: (0, c)),
            pl.BlockSpec((BS, 1, C, D), lambda c: (0, c, 0, 0)),
            pl.BlockSpec((1, C, H_MOD, MOD_IN), lambda c: (c, 0, 0, 0)),
            pl.BlockSpec((C, H_MOD), lambda c: (c, 0)),
            pl.BlockSpec((1, C, H_MOD, MOD_OUT), lambda c: (c, 0, 0, 0)),
            pl.BlockSpec((C, MOD_OUT), lambda c: (c, 0)),
            pl.BlockSpec((1, C, D), lambda c: (c, 0, 0)),
            pl.BlockSpec((1, 1, B_BORDER), lambda c: (c, 0, 0)),
        ],
        out_specs=[
            pl.BlockSpec((BS, 1, C, K), lambda c: (0, c, 0, 0)),
            pl.BlockSpec((BS, 1, B_BORDER, K_B), lambda c: (0, c, 0, 0)),
            pl.BlockSpec((BS, C), lambda c: (0, c)),
            pl.BlockSpec((BS, 1, C, D), lambda c: (0, c, 0, 0)),
        ],
        out_shape=[
            jax.ShapeDtypeStruct((BS, NC, C, K), jnp.float32),
            jax.ShapeDtypeStruct((BS, NC, B_BORDER, K_B), jnp.float32),
            jax.ShapeDtypeStruct((BS, N), jnp.float32),
            jax.ShapeDtypeStruct((BS, NC, C, D), jnp.float32),
        ],
        compiler_params=pltpu.CompilerParams(
            dimension_semantics=("arbitrary",),
        ),
    )(hebbian_traces, h, decay_logit.reshape(BS, N), primitives,
      mod_w1.astype(jnp.bfloat16).reshape(NC, C, H_MOD, MOD_IN), mod_b1,
      mod_w2.astype(jnp.bfloat16).reshape(NC, C, H_MOD, MOD_OUT),
      mod_b2, neuron_id, bidx)

    return (wconn, sel, ndec.reshape(BS, NC, C), nprim)
